# Initial kernel scaffold; baseline (speedup 1.0000x reference)
import jax
import jax.numpy as jnp
from jax.experimental import pallas as pl


def kernel(x, emb, w_ih1, w_hh1, b_ih1, b_hh1, w_ih2, w_hh2, b_ih2, b_hh2, fc_w, fc_b):
    raise NotImplementedError("write your pallas kernel here")


# speedup vs baseline: 5.1482x; 5.1482x over previous
import functools

import jax
import jax.numpy as jnp
from jax import lax
from jax.experimental import pallas as pl
from jax.experimental.pallas import tpu as pltpu
from jax.experimental.pallas import tpu_sc as plsc

VOCAB = 110000
EMB = 64
H = 128
NCLS = 20
B = 1024
L = 200

TOTAL = B * L
NW = 32
PER_W = TOTAL // NW
CH = 128
NCH = PER_W // CH

LT = 8
NLC = L // LT


def _sc_gather_body(table_hbm, idx_hbm, out_hbm, idx_v, rows_v, sem):
    wid = lax.axis_index("s") * 2 + lax.axis_index("c")
    pltpu.sync_copy(idx_hbm.at[wid], idx_v)

    def body(j, _):
        pltpu.async_copy(table_hbm.at[idx_v.at[j]], rows_v, sem).wait()
        pltpu.sync_copy(rows_v, out_hbm.at[pl.ds((wid * NCH + j) * CH, CH)])
        return 0

    lax.fori_loop(0, NCH, body, 0)


@functools.cache
def _sc_gather():
    return pl.kernel(
        _sc_gather_body,
        out_type=jax.ShapeDtypeStruct((TOTAL, EMB), jnp.float32),
        mesh=plsc.VectorSubcoreMesh(core_axis_name="c", subcore_axis_name="s"),
        scratch_types=[
            pltpu.VMEM((NCH, CH), jnp.int32),
            pltpu.VMEM((CH, EMB), jnp.float32),
            pltpu.SemaphoreType.DMA,
        ],
        compiler_params=pltpu.CompilerParams(use_tc_tiling_on_sc=False),
    )


def _rnn_body(e_ref, w1_ref, wh1_ref, w2_ref, wh2_ref, fct_ref,
              b1_ref, b2_ref, fcb_ref, out_ref, h1_ref, h2_ref, h1buf_ref):
    lc = pl.program_id(0)

    @pl.when(lc == 0)
    def _():
        h1_ref[...] = jnp.zeros_like(h1_ref)
        h2_ref[...] = jnp.zeros_like(h2_ref)

    e = e_ref[...].reshape(LT * B, EMB)
    xp1 = jnp.dot(e, w1_ref[...], preferred_element_type=jnp.float32)
    xp1 = xp1 + b1_ref[...]

    h1 = h1_ref[...]
    for t in range(LT):
        h1 = jnp.tanh(
            xp1[t * B:(t + 1) * B]
            + jnp.dot(h1, wh1_ref[...], preferred_element_type=jnp.float32))
        h1buf_ref[t * B:(t + 1) * B] = h1
    h1_ref[...] = h1

    xp2 = jnp.dot(h1buf_ref[...], w2_ref[...],
                  preferred_element_type=jnp.float32)
    xp2 = xp2 + b2_ref[...]

    h2 = h2_ref[...]
    for t in range(LT):
        h2 = jnp.tanh(
            xp2[t * B:(t + 1) * B]
            + jnp.dot(h2, wh2_ref[...], preferred_element_type=jnp.float32))
    h2_ref[...] = h2

    @pl.when(lc == NLC - 1)
    def _():
        out_ref[...] = (
            jnp.dot(h2, fct_ref[...], preferred_element_type=jnp.float32)
            + fcb_ref[...])


_rnn_call = pl.pallas_call(
    _rnn_body,
    grid=(NLC,),
    in_specs=[
        pl.BlockSpec((LT, B, EMB), lambda l: (l, 0, 0)),
        pl.BlockSpec((EMB, H), lambda l: (0, 0)),
        pl.BlockSpec((H, H), lambda l: (0, 0)),
        pl.BlockSpec((H, H), lambda l: (0, 0)),
        pl.BlockSpec((H, H), lambda l: (0, 0)),
        pl.BlockSpec((H, NCLS), lambda l: (0, 0)),
        pl.BlockSpec((1, H), lambda l: (0, 0)),
        pl.BlockSpec((1, H), lambda l: (0, 0)),
        pl.BlockSpec((1, NCLS), lambda l: (0, 0)),
    ],
    out_specs=pl.BlockSpec((B, NCLS), lambda l: (0, 0)),
    out_shape=jax.ShapeDtypeStruct((B, NCLS), jnp.float32),
    scratch_shapes=[
        pltpu.VMEM((B, H), jnp.float32),
        pltpu.VMEM((B, H), jnp.float32),
        pltpu.VMEM((LT * B, H), jnp.float32),
    ],
    compiler_params=pltpu.CompilerParams(
        dimension_semantics=("arbitrary",)),
)


def kernel(x, emb, w_ih1, w_hh1, b_ih1, b_hh1,
           w_ih2, w_hh2, b_ih2, b_hh2, fc_w, fc_b):
    idx3d = x.T.reshape(NW, NCH, CH).astype(jnp.int32)
    e = _sc_gather()(emb, idx3d)
    e3 = e.reshape(L, B, EMB)
    out = _rnn_call(
        e3,
        w_ih1.T, w_hh1.T, w_ih2.T, w_hh2.T, fc_w.T,
        (b_ih1 + b_hh1)[None, :], (b_ih2 + b_hh2)[None, :], fc_b[None, :])
    return out


# speedup vs baseline: 6.1656x; 1.1976x over previous
import functools

import jax
import jax.numpy as jnp
from jax import lax
from jax.experimental import pallas as pl
from jax.experimental.pallas import tpu as pltpu
from jax.experimental.pallas import tpu_sc as plsc

VOCAB = 110000
EMB = 64
H = 128
NCLS = 20
B = 1024
L = 200

TOTAL = B * L
NW = 32
PER_W = TOTAL // NW
CH = 128
NCH = PER_W // CH
NCHP = 56

BM = 5000
NMC = VOCAB // BM

LT = 8
NLC = L // LT


def _proj_body(e_ref, w_ref, b_ref, p_ref):
    p_ref[...] = (
        jnp.dot(e_ref[...], w_ref[...], preferred_element_type=jnp.float32)
        + b_ref[...])


_proj_call = pl.pallas_call(
    _proj_body,
    grid=(NMC,),
    in_specs=[
        pl.BlockSpec((BM, EMB), lambda i: (i, 0)),
        pl.BlockSpec((EMB, H), lambda i: (0, 0)),
        pl.BlockSpec((1, H), lambda i: (0, 0)),
    ],
    out_specs=pl.BlockSpec((BM, H), lambda i: (i, 0)),
    out_shape=jax.ShapeDtypeStruct((VOCAB, H), jnp.float32),
)


def _sc_gather_body(table_hbm, idx_hbm, out_hbm, idx_v, rows_v, sem):
    wid = lax.axis_index("s") * 2 + lax.axis_index("c")
    pltpu.sync_copy(idx_hbm.at[wid], idx_v)

    def body(j, _):
        pltpu.async_copy(table_hbm.at[idx_v.at[j]], rows_v, sem).wait()
        pltpu.sync_copy(rows_v, out_hbm.at[pl.ds((wid * NCH + j) * CH, CH)])
        return 0

    lax.fori_loop(0, NCH, body, 0)


@functools.cache
def _sc_gather():
    return pl.kernel(
        _sc_gather_body,
        out_type=jax.ShapeDtypeStruct((TOTAL, H), jnp.float32),
        mesh=plsc.VectorSubcoreMesh(core_axis_name="c", subcore_axis_name="s"),
        scratch_types=[
            pltpu.VMEM((NCHP, CH), jnp.int32),
            pltpu.VMEM((CH, H), jnp.float32),
            pltpu.SemaphoreType.DMA,
        ],
        compiler_params=pltpu.CompilerParams(use_tc_tiling_on_sc=False),
    )


def _rnn_body(xp1_ref, wh1_ref, w2_ref, wh2_ref, fct_ref,
              b2_ref, fcb_ref, out_ref, h1_ref, h2_ref, h1buf_ref):
    lc = pl.program_id(0)

    @pl.when(lc == 0)
    def _():
        h1_ref[...] = jnp.zeros_like(h1_ref)
        h2_ref[...] = jnp.zeros_like(h2_ref)

    h1 = h1_ref[...]
    for t in range(LT):
        h1 = jnp.tanh(
            xp1_ref[t * B:(t + 1) * B]
            + jnp.dot(h1, wh1_ref[...], preferred_element_type=jnp.float32))
        h1buf_ref[t * B:(t + 1) * B] = h1
    h1_ref[...] = h1

    xp2 = jnp.dot(h1buf_ref[...], w2_ref[...],
                  preferred_element_type=jnp.float32)
    xp2 = xp2 + b2_ref[...]

    h2 = h2_ref[...]
    for t in range(LT):
        h2 = jnp.tanh(
            xp2[t * B:(t + 1) * B]
            + jnp.dot(h2, wh2_ref[...], preferred_element_type=jnp.float32))
    h2_ref[...] = h2

    @pl.when(lc == NLC - 1)
    def _():
        out_ref[...] = (
            jnp.dot(h2, fct_ref[...], preferred_element_type=jnp.float32)
            + fcb_ref[...])


_rnn_call = pl.pallas_call(
    _rnn_body,
    grid=(NLC,),
    in_specs=[
        pl.BlockSpec((LT * B, H), lambda l: (l, 0)),
        pl.BlockSpec((H, H), lambda l: (0, 0)),
        pl.BlockSpec((H, H), lambda l: (0, 0)),
        pl.BlockSpec((H, H), lambda l: (0, 0)),
        pl.BlockSpec((H, NCLS), lambda l: (0, 0)),
        pl.BlockSpec((1, H), lambda l: (0, 0)),
        pl.BlockSpec((1, NCLS), lambda l: (0, 0)),
    ],
    out_specs=pl.BlockSpec((B, NCLS), lambda l: (0, 0)),
    out_shape=jax.ShapeDtypeStruct((B, NCLS), jnp.float32),
    scratch_shapes=[
        pltpu.VMEM((B, H), jnp.float32),
        pltpu.VMEM((B, H), jnp.float32),
        pltpu.VMEM((LT * B, H), jnp.float32),
    ],
    compiler_params=pltpu.CompilerParams(
        dimension_semantics=("arbitrary",)),
)


def kernel(x, emb, w_ih1, w_hh1, b_ih1, b_hh1,
           w_ih2, w_hh2, b_ih2, b_hh2, fc_w, fc_b):
    p = _proj_call(emb, w_ih1.T, (b_ih1 + b_hh1)[None, :])

    idx3d = jnp.pad(x.T.reshape(NW, NCH, CH).astype(jnp.int32),
                    ((0, 0), (0, NCHP - NCH), (0, 0)))
    xp1 = _sc_gather()(p, idx3d)

    out = _rnn_call(
        xp1,
        w_hh1.T, w_ih2.T, w_hh2.T, fc_w.T,
        (b_ih2 + b_hh2)[None, :], fc_b[None, :])
    return out
